# baseline (speedup 1.0000x reference)
import functools

import jax
import jax.numpy as jnp
from jax.experimental import pallas as pl
from jax.experimental.pallas import tpu as pltpu


_NT = (((1,), (1,)), ((), ()))


def _ffn_kernel(ned, x_ref, wg_ref, wup_ref, wdn_ref, o_ref, scale_ref, xb_ref):
    ed = pl.program_id(1)

    @pl.when(ed == 0)
    def _cast():
        xb_ref[...] = x_ref[...].astype(jnp.bfloat16)
    xb = xb_ref[...]

    @pl.when(ed == 0)
    def _gate():
        logits = jax.lax.dot_general(xb, wg_ref[...], _NT,
                                     preferred_element_type=jnp.float32)
        gp = jax.nn.softmax(logits, axis=-1)
        m1 = jnp.max(gp, axis=-1, keepdims=True)
        ne = gp.shape[-1]
        iota = jax.lax.broadcasted_iota(jnp.int32, gp.shape, 1)
        amax = jnp.min(jnp.where(gp == m1, iota, ne), axis=-1, keepdims=True)
        m2 = jnp.max(jnp.where(iota == amax, -jnp.inf, gp),
                     axis=-1, keepdims=True)
        s = m1 + m2
        scale_ref[...] = s / (s + 1e-9)

    h = jax.lax.dot_general(xb, wup_ref[...], _NT,
                            preferred_element_type=jnp.float32)
    h = (h * jax.nn.sigmoid(h)).astype(jnp.bfloat16)
    contrib = jax.lax.dot_general(h, wdn_ref[...], _NT,
                                  preferred_element_type=jnp.float32)

    @pl.when(ed == 0)
    def _init():
        o_ref[...] = contrib

    @pl.when(ed != 0)
    def _acc():
        o_ref[...] += contrib

    @pl.when(ed == ned - 1)
    def _fin():
        o_ref[...] *= scale_ref[...]


def kernel(x, W_gate, W_up, W_down):
    B, S, D = x.shape
    T = B * S
    ED = W_up.shape[0]
    NE = W_gate.shape[0]

    xf = x.reshape(T, D)
    wg = W_gate.astype(jnp.bfloat16)
    wup = W_up.astype(jnp.bfloat16)
    wdn = W_down.astype(jnp.bfloat16)

    TM = 256 if T % 256 == 0 else T
    EB = 2048 if ED % 2048 == 0 else ED
    ntm, ned = T // TM, ED // EB

    out = pl.pallas_call(
        functools.partial(_ffn_kernel, ned),
        grid=(ntm, ned),
        in_specs=[
            pl.BlockSpec((TM, D), lambda i, j: (i, 0)),
            pl.BlockSpec((NE, D), lambda i, j: (0, 0)),
            pl.BlockSpec((EB, D), lambda i, j: (j, 0)),
            pl.BlockSpec((D, EB), lambda i, j: (0, j)),
        ],
        out_specs=pl.BlockSpec((TM, D), lambda i, j: (i, 0)),
        out_shape=jax.ShapeDtypeStruct((T, D), jnp.float32),
        scratch_shapes=[
            pltpu.VMEM((TM, 1), jnp.float32),
            pltpu.VMEM((TM, D), jnp.bfloat16),
        ],
        compiler_params=pltpu.CompilerParams(
            dimension_semantics=("parallel", "arbitrary")),
    )(xf, wg, wup, wdn)
    return out.reshape(B, S, D)


# speedup vs baseline: 1.1283x; 1.1283x over previous
import functools

import jax
import jax.numpy as jnp
from jax.experimental import pallas as pl
from jax.experimental.pallas import tpu as pltpu


_NT = (((1,), (1,)), ((), ()))


def _ffn_kernel(ned, x_ref, wg_ref, wup_ref, wdn_ref, o_ref, scale_ref, xb_ref):
    ed = pl.program_id(1)

    @pl.when(ed == 0)
    def _gate():
        xb0 = x_ref[...].astype(jnp.bfloat16)
        xb_ref[...] = xb0
        logits = jax.lax.dot_general(xb0, wg_ref[...], _NT,
                                     preferred_element_type=jnp.float32)
        gp = jax.nn.softmax(logits, axis=-1)
        m1 = jnp.max(gp, axis=-1, keepdims=True)
        ne = gp.shape[-1]
        iota = jax.lax.broadcasted_iota(jnp.int32, gp.shape, 1)
        amax = jnp.min(jnp.where(gp == m1, iota, ne), axis=-1, keepdims=True)
        m2 = jnp.max(jnp.where(iota == amax, -jnp.inf, gp),
                     axis=-1, keepdims=True)
        s = m1 + m2
        scale_ref[...] = s / (s + 1e-9)

    h = jax.lax.dot_general(xb_ref[...], wup_ref[...], _NT,
                            preferred_element_type=jnp.float32)
    h = (h * jax.nn.sigmoid(h) * scale_ref[...]).astype(jnp.bfloat16)
    contrib = jax.lax.dot_general(h, wdn_ref[...], _NT,
                                  preferred_element_type=jnp.float32)

    @pl.when(ed == 0)
    def _init():
        o_ref[...] = contrib

    @pl.when(ed != 0)
    def _acc():
        o_ref[...] += contrib


def kernel(x, W_gate, W_up, W_down):
    B, S, D = x.shape
    T = B * S
    ED = W_up.shape[0]
    NE = W_gate.shape[0]

    xf = x.reshape(T, D)
    wg = W_gate.astype(jnp.bfloat16)
    wup = W_up.astype(jnp.bfloat16)
    wdn = W_down.astype(jnp.bfloat16)

    TM = 512 if T % 512 == 0 else T
    EB = 2048 if ED % 2048 == 0 else ED
    ntm, ned = T // TM, ED // EB

    out = pl.pallas_call(
        functools.partial(_ffn_kernel, ned),
        grid=(ntm, ned),
        in_specs=[
            pl.BlockSpec((TM, D), lambda i, j: (i, 0)),
            pl.BlockSpec((NE, D), lambda i, j: (0, 0)),
            pl.BlockSpec((EB, D), lambda i, j: (j, 0)),
            pl.BlockSpec((D, EB), lambda i, j: (0, j)),
        ],
        out_specs=pl.BlockSpec((TM, D), lambda i, j: (i, 0)),
        out_shape=jax.ShapeDtypeStruct((T, D), jnp.float32),
        scratch_shapes=[
            pltpu.VMEM((TM, 1), jnp.float32),
            pltpu.VMEM((TM, D), jnp.bfloat16),
        ],
        compiler_params=pltpu.CompilerParams(
            dimension_semantics=("parallel", "arbitrary")),
    )(xf, wg, wup, wdn)
    return out.reshape(B, S, D)


# speedup vs baseline: 1.1294x; 1.0010x over previous
import functools

import jax
import jax.numpy as jnp
from jax.experimental import pallas as pl
from jax.experimental.pallas import tpu as pltpu


_NT = (((1,), (1,)), ((), ()))


def _ffn_kernel(ned, x_ref, wg_ref, wup_ref, wdn_ref, o_ref, scale_ref, xb_ref,
                hb_ref):
    ed = pl.program_id(1)

    @pl.when(ed == 0)
    def _gate():
        xb0 = x_ref[...].astype(jnp.bfloat16)
        xb_ref[...] = xb0
        logits = jax.lax.dot_general(xb0, wg_ref[...], _NT,
                                     preferred_element_type=jnp.float32)
        gp = jax.nn.softmax(logits, axis=-1)
        m1 = jnp.max(gp, axis=-1, keepdims=True)
        ne = gp.shape[-1]
        iota = jax.lax.broadcasted_iota(jnp.int32, gp.shape, 1)
        amax = jnp.min(jnp.where(gp == m1, iota, ne), axis=-1, keepdims=True)
        m2 = jnp.max(jnp.where(iota == amax, -jnp.inf, gp),
                     axis=-1, keepdims=True)
        s = m1 + m2
        scale_ref[...] = s / (s + 1e-9)

    eb = wup_ref.shape[0]
    nch = 4
    ch = eb // nch
    xb = xb_ref[...]
    s = scale_ref[...]
    for c in range(nch):
        hc = jax.lax.dot_general(xb, wup_ref[c * ch:(c + 1) * ch, :], _NT,
                                 preferred_element_type=jnp.float32)
        hb_ref[:, c * ch:(c + 1) * ch] = \
            (hc * jax.nn.sigmoid(hc) * s).astype(jnp.bfloat16)
    contrib = jax.lax.dot_general(hb_ref[...], wdn_ref[...], _NT,
                                  preferred_element_type=jnp.float32)

    @pl.when(ed == 0)
    def _init():
        o_ref[...] = contrib

    @pl.when(ed != 0)
    def _acc():
        o_ref[...] += contrib


def kernel(x, W_gate, W_up, W_down):
    B, S, D = x.shape
    T = B * S
    ED = W_up.shape[0]
    NE = W_gate.shape[0]

    xf = x.reshape(T, D)
    wg = W_gate.astype(jnp.bfloat16)
    wup = W_up.astype(jnp.bfloat16)
    wdn = W_down.astype(jnp.bfloat16)

    TM = 512 if T % 512 == 0 else T
    EB = 2048 if ED % 2048 == 0 else ED
    ntm, ned = T // TM, ED // EB

    out = pl.pallas_call(
        functools.partial(_ffn_kernel, ned),
        grid=(ntm, ned),
        in_specs=[
            pl.BlockSpec((TM, D), lambda i, j: (i, 0)),
            pl.BlockSpec((NE, D), lambda i, j: (0, 0)),
            pl.BlockSpec((EB, D), lambda i, j: (j, 0)),
            pl.BlockSpec((D, EB), lambda i, j: (0, j)),
        ],
        out_specs=pl.BlockSpec((TM, D), lambda i, j: (i, 0)),
        out_shape=jax.ShapeDtypeStruct((T, D), jnp.float32),
        scratch_shapes=[
            pltpu.VMEM((TM, 1), jnp.float32),
            pltpu.VMEM((TM, D), jnp.bfloat16),
            pltpu.VMEM((TM, EB), jnp.bfloat16),
        ],
        compiler_params=pltpu.CompilerParams(
            dimension_semantics=("parallel", "arbitrary")),
    )(xf, wg, wup, wdn)
    return out.reshape(B, S, D)
